# baseline (speedup 1.0000x reference)
import functools

import jax
import jax.numpy as jnp
from jax.experimental import pallas as pl
from jax.experimental.pallas import tpu as pltpu

N = 10000
D = 128


def _mlp_body(z_ref, a_ref, w1_ref, b1_ref, g_ref, be_ref, w2_ref, b2_ref, out_ref):
    h = z_ref[0] + a_ref[0]
    h = jnp.dot(h, w1_ref[...].T, preferred_element_type=jnp.float32) + b1_ref[...]
    mu = jnp.mean(h, axis=0, keepdims=True)
    var = jnp.mean((h - mu) ** 2, axis=0, keepdims=True)
    h = g_ref[...] * (h - mu) * jax.lax.rsqrt(var + 1e-5) + be_ref[...]
    h = jnp.maximum(h, 0.0)
    h = jnp.dot(h, w2_ref[...].T, preferred_element_type=jnp.float32) + b2_ref[...]
    out_ref[0] = jnp.maximum(h, 0.0)


def _mlp_pair(z2, a2, w1, b1, g, be, w2, b2):
    grid = (2,)
    blk = pl.BlockSpec((1, N, D), lambda i: (i, 0, 0))
    wspec = pl.BlockSpec((D, D), lambda i: (0, 0))
    vspec = pl.BlockSpec((D,), lambda i: (0,))
    return pl.pallas_call(
        _mlp_body,
        grid=grid,
        in_specs=[blk, blk, wspec, vspec, vspec, vspec, wspec, vspec],
        out_specs=blk,
        out_shape=jax.ShapeDtypeStruct((2, N, D), jnp.float32),
    )(z2, a2, w1, b1, g, be, w2, b2)


def _readout_body(zr_ref, zp_ref, wd_ref, out_ref):
    zr = zr_ref[...]
    zp = zp_ref[...]
    s = jax.nn.sigmoid(jnp.mean(zr, axis=0, keepdims=True))
    w = jnp.dot(wd_ref[...].T, s.T, preferred_element_type=jnp.float32)
    real = jax.nn.sigmoid(jnp.dot(zr, w, preferred_element_type=jnp.float32))
    pert = jax.nn.sigmoid(jnp.dot(zp, w, preferred_element_type=jnp.float32))
    eps = 1e-12
    real = jnp.clip(real, eps, 1.0 - eps)
    pert = jnp.clip(pert, eps, 1.0 - eps)
    total = jnp.sum(jnp.log(real)) + jnp.sum(jnp.log(1.0 - pert))
    out_ref[...] = jnp.broadcast_to(-total / (2.0 * N), (1, 1))


def _readout(zr, zp, wd):
    return pl.pallas_call(
        _readout_body,
        out_shape=jax.ShapeDtypeStruct((1, 1), jnp.float32),
    )(zr, zp, wd)


def kernel(x, ei, perm, W1a, b1a, g1a, be1a, W2a, b2a, W1b, b1b, g1b, be1b, W2b, b2b, Wd):
    src = ei[0]
    dst = ei[1]
    xp = x[perm]
    z2 = jnp.stack([x, xp])

    def aggr_pair(z2):
        zeros = jnp.zeros((N, D), jnp.float32)
        ar = zeros.at[dst].add(z2[0][src])
        ap = zeros.at[dst].add(z2[1][src])
        return jnp.stack([ar, ap])

    a1 = aggr_pair(z2)
    h1 = _mlp_pair(z2, a1, W1a, b1a, g1a, be1a, W2a, b2a)
    a2 = aggr_pair(h1)
    h2 = _mlp_pair(h1, a2, W1b, b1b, g1b, be1b, W2b, b2b)
    loss = _readout(h2[0], h2[1], Wd)
    return loss.reshape(())


# speedup vs baseline: 1.4155x; 1.4155x over previous
import functools

import jax
import jax.numpy as jnp
from jax import lax
from jax.experimental import pallas as pl
from jax.experimental.pallas import tpu as pltpu
from jax.experimental.pallas import tpu_sc as plsc

N = 10000
D = 128
E = 320000

NSUB = 16
CHUNK = 128
EPS = E // NSUB
IBLK = 16
IB = 10
CH = IB * IBLK
SLOTS = CH * CHUNK
NPAD = 10240
RPS = NPAD // NSUB


def _aggr_body(zflat, srcp, dstp, zrows, out, src_v, dst_v, rows_v, acc, sem):
    c = lax.axis_index("c")
    s = lax.axis_index("s")
    pltpu.sync_copy(zrows, acc.at[pl.ds(s * RPS, RPS)])
    plsc.subcore_barrier()

    def outer(ib, carry):
        pltpu.sync_copy(srcp.at[c, s, ib], src_v)
        pltpu.sync_copy(dstp.at[s, ib], dst_v)

        def body(j, carry2):
            pltpu.async_copy(zflat.at[src_v.at[j]], rows_v, sem).wait()
            pltpu.sync_copy(rows_v, acc.at[dst_v.at[j]], add=True)
            return carry2

        return lax.fori_loop(0, IBLK, body, carry)

    lax.fori_loop(0, IB, outer, jnp.int32(0))
    plsc.subcore_barrier()
    pltpu.sync_copy(acc.at[pl.ds(s * RPS, RPS)], out.at[c, pl.ds(s * RPS, RPS)])


def _aggr_pair(zflat, srcp, dstp, zrows):
    mesh = plsc.VectorSubcoreMesh(core_axis_name="c", subcore_axis_name="s")
    f = functools.partial(
        pl.kernel,
        out_type=jax.ShapeDtypeStruct((2, NPAD, D), jnp.float32),
        mesh=mesh,
        scratch_types=[
            pltpu.VMEM((IBLK, CHUNK), jnp.int32),
            pltpu.VMEM((IBLK, CHUNK), jnp.int32),
            pltpu.VMEM((CHUNK, D), jnp.float32),
            pltpu.VMEM_SHARED((NPAD, D), jnp.float32),
            pltpu.SemaphoreType.DMA,
        ],
    )(_aggr_body)
    return f(zflat, srcp, dstp, zrows)


def _mlp_body(z_ref, a_ref, w1_ref, b1_ref, g_ref, be_ref, w2_ref, b2_ref, out_ref):
    h = z_ref[0] + a_ref[0]
    h = jnp.dot(h, w1_ref[...].T, preferred_element_type=jnp.float32) + b1_ref[...]
    mu = jnp.mean(h, axis=0, keepdims=True)
    var = jnp.mean((h - mu) ** 2, axis=0, keepdims=True)
    h = g_ref[...] * (h - mu) * lax.rsqrt(var + 1e-5) + be_ref[...]
    h = jnp.maximum(h, 0.0)
    h = jnp.dot(h, w2_ref[...].T, preferred_element_type=jnp.float32) + b2_ref[...]
    out_ref[0] = jnp.maximum(h, 0.0)


def _mlp_pair(z2, a2, w1, b1, g, be, w2, b2):
    blk = pl.BlockSpec((1, N, D), lambda i: (i, 0, 0))
    wspec = pl.BlockSpec((D, D), lambda i: (0, 0))
    vspec = pl.BlockSpec((D,), lambda i: (0,))
    return pl.pallas_call(
        _mlp_body,
        grid=(2,),
        in_specs=[blk, blk, wspec, vspec, vspec, vspec, wspec, vspec],
        out_specs=blk,
        out_shape=jax.ShapeDtypeStruct((2, N, D), jnp.float32),
    )(z2, a2, w1, b1, g, be, w2, b2)


def _readout_body(zr_ref, zp_ref, wd_ref, out_ref):
    zr = zr_ref[...]
    zp = zp_ref[...]
    s = jax.nn.sigmoid(jnp.mean(zr, axis=0, keepdims=True))
    w = jnp.dot(wd_ref[...].T, s.T, preferred_element_type=jnp.float32)
    real = jax.nn.sigmoid(jnp.dot(zr, w, preferred_element_type=jnp.float32))
    pert = jax.nn.sigmoid(jnp.dot(zp, w, preferred_element_type=jnp.float32))
    eps = 1e-12
    real = jnp.clip(real, eps, 1.0 - eps)
    pert = jnp.clip(pert, eps, 1.0 - eps)
    total = jnp.sum(jnp.log(real)) + jnp.sum(jnp.log(1.0 - pert))
    out_ref[...] = jnp.broadcast_to(-total / (2.0 * N), (1, 1))


def _readout(zr, zp, wd):
    return pl.pallas_call(
        _readout_body,
        out_shape=jax.ShapeDtypeStruct((1, 1), jnp.float32),
    )(zr, zp, wd)


def kernel(x, ei, perm, W1a, b1a, g1a, be1a, W2a, b2a, W1b, b1b, g1b, be1b, W2b, b2b, Wd):
    src = ei[0]
    dst = ei[1]
    perm = perm.astype(jnp.int32)

    pad = SLOTS - EPS
    src_r = src.reshape(NSUB, EPS)
    dst_r = dst.reshape(NSUB, EPS)
    zero_pad = jnp.zeros((NSUB, pad), jnp.int32)
    src_pad = jnp.concatenate([src_r, zero_pad], axis=1)
    dummy = N + (jnp.arange(pad, dtype=jnp.int32) % (NPAD - N))
    dst_pad = jnp.concatenate([dst_r, jnp.broadcast_to(dummy, (NSUB, pad))], axis=1)
    dstp = dst_pad.reshape(NSUB, IB, IBLK, CHUNK)
    sp_r = perm[src].reshape(NSUB, EPS)
    sp_pad = jnp.concatenate([sp_r, zero_pad], axis=1)
    srcp1 = jnp.stack([src_pad, sp_pad]).reshape(2, NSUB, IB, IBLK, CHUNK)
    srcp2 = jnp.stack([src_pad, src_pad + N]).reshape(2, NSUB, IB, IBLK, CHUNK)
    zrows = jnp.zeros((RPS, D), jnp.float32)

    xp = x[perm]
    z1 = jnp.stack([x, xp])

    a1 = _aggr_pair(x, srcp1, dstp, zrows)
    h1 = _mlp_pair(z1, a1[:, :N], W1a, b1a, g1a, be1a, W2a, b2a)
    a2 = _aggr_pair(h1.reshape(2 * N, D), srcp2, dstp, zrows)
    h2 = _mlp_pair(h1, a2[:, :N], W1b, b1b, g1b, be1b, W2b, b2b)
    loss = _readout(h2[0], h2[1], Wd)
    return loss.reshape(())


# speedup vs baseline: 3.1644x; 2.2355x over previous
import functools

import jax
import jax.numpy as jnp
from jax import lax
from jax.experimental import pallas as pl
from jax.experimental.pallas import tpu as pltpu
from jax.experimental.pallas import tpu_sc as plsc

N = 10000
D = 128
E = 320000

NSUB = 16
CHUNK = 128
EPS = E // NSUB
IBLK = 16
IB = 10
CH = IB * IBLK
SLOTS = CH * CHUNK
NF = 10240
NPAD = 10240
RPS = NPAD // NSUB
PRW = NF // NSUB
PCH = 64
PCC = PRW // PCH
CPW = 624


def _perm_body(x_hbm, permp, out, perm_v, rows_v, sem):
    c = lax.axis_index("c")
    s = lax.axis_index("s")

    @pl.when(c == 0)
    def _():
        pltpu.sync_copy(x_hbm.at[pl.ds(s * CPW, CPW)], out.at[pl.ds(s * CPW, CPW)])

        @pl.when(s == 0)
        def _():
            pltpu.sync_copy(x_hbm.at[pl.ds(NSUB * CPW, N - NSUB * CPW)],
                            out.at[pl.ds(NSUB * CPW, N - NSUB * CPW)])

    @pl.when(c == 1)
    def _():
        pltpu.sync_copy(permp.at[s], perm_v)
        base = NF + s * PRW
        for cc in range(PCC):
            pltpu.async_copy(x_hbm.at[perm_v.at[cc]], rows_v, sem).wait()
            pltpu.sync_copy(rows_v, out.at[pl.ds(base + cc * PCH, PCH)])


def _perm_stack(x, permp):
    mesh = plsc.VectorSubcoreMesh(core_axis_name="c", subcore_axis_name="s")
    f = functools.partial(
        pl.kernel,
        out_type=jax.ShapeDtypeStruct((2 * NF, D), jnp.float32),
        mesh=mesh,
        scratch_types=[
            pltpu.VMEM((PCC, PCH), jnp.int32),
            pltpu.VMEM((PCH, D), jnp.float32),
            pltpu.SemaphoreType.DMA,
        ],
    )(_perm_body)
    return f(x, permp)


def _aggr_body(zflat, srcp, dstp, zrows, out, src_v, dst_v, rows_v, acc, sem):
    c = lax.axis_index("c")
    s = lax.axis_index("s")
    pltpu.sync_copy(zrows, acc.at[pl.ds(s * RPS, RPS)])
    plsc.subcore_barrier()

    def outer(ib, carry):
        pltpu.sync_copy(srcp.at[c, s, ib], src_v)
        pltpu.sync_copy(dstp.at[s, ib], dst_v)

        def body(j, carry2):
            pltpu.async_copy(zflat.at[src_v.at[j]], rows_v, sem).wait()
            pltpu.sync_copy(rows_v, acc.at[dst_v.at[j]], add=True)
            return carry2

        return lax.fori_loop(0, IBLK, body, carry)

    lax.fori_loop(0, IB, outer, jnp.int32(0))
    plsc.subcore_barrier()
    pltpu.sync_copy(acc.at[pl.ds(s * RPS, RPS)], out.at[c, pl.ds(s * RPS, RPS)])


def _aggr_pair(zflat, srcp, dstp, zrows):
    mesh = plsc.VectorSubcoreMesh(core_axis_name="c", subcore_axis_name="s")
    f = functools.partial(
        pl.kernel,
        out_type=jax.ShapeDtypeStruct((2, NPAD, D), jnp.float32),
        mesh=mesh,
        scratch_types=[
            pltpu.VMEM((IBLK, CHUNK), jnp.int32),
            pltpu.VMEM((IBLK, CHUNK), jnp.int32),
            pltpu.VMEM((CHUNK, D), jnp.float32),
            pltpu.VMEM_SHARED((NPAD, D), jnp.float32),
            pltpu.SemaphoreType.DMA,
        ],
    )(_aggr_body)
    return f(zflat, srcp, dstp, zrows)


def _mlp_body(z_ref, a_ref, w1_ref, b1_ref, g_ref, be_ref, w2_ref, b2_ref, out_ref):
    h = z_ref[0] + a_ref[0]
    h = jnp.dot(h, w1_ref[...].T, preferred_element_type=jnp.float32) + b1_ref[...]
    mu = jnp.mean(h, axis=0, keepdims=True)
    var = jnp.mean((h - mu) ** 2, axis=0, keepdims=True)
    h = g_ref[...] * (h - mu) * lax.rsqrt(var + 1e-5) + be_ref[...]
    h = jnp.maximum(h, 0.0)
    h = jnp.dot(h, w2_ref[...].T, preferred_element_type=jnp.float32) + b2_ref[...]
    out_ref[0] = jnp.maximum(h, 0.0)


def _mlp_pair(z2, a2, w1, b1, g, be, w2, b2):
    nz = z2.shape[1]
    zblk = pl.BlockSpec((1, N, D), lambda i: (i, 0, 0))
    wspec = pl.BlockSpec((D, D), lambda i: (0, 0))
    vspec = pl.BlockSpec((D,), lambda i: (0,))
    return pl.pallas_call(
        _mlp_body,
        grid=(2,),
        in_specs=[zblk, zblk, wspec, vspec, vspec, vspec, wspec, vspec],
        out_specs=pl.BlockSpec((1, N, D), lambda i: (i, 0, 0)),
        out_shape=jax.ShapeDtypeStruct((2, N, D), jnp.float32),
    )(z2, a2, w1, b1, g, be, w2, b2)


def _readout_body(zr_ref, zp_ref, wd_ref, out_ref):
    zr = zr_ref[...]
    zp = zp_ref[...]
    s = jax.nn.sigmoid(jnp.mean(zr, axis=0, keepdims=True))
    w = jnp.dot(wd_ref[...].T, s.T, preferred_element_type=jnp.float32)
    real = jax.nn.sigmoid(jnp.dot(zr, w, preferred_element_type=jnp.float32))
    pert = jax.nn.sigmoid(jnp.dot(zp, w, preferred_element_type=jnp.float32))
    eps = 1e-12
    real = jnp.clip(real, eps, 1.0 - eps)
    pert = jnp.clip(pert, eps, 1.0 - eps)
    total = jnp.sum(jnp.log(real)) + jnp.sum(jnp.log(1.0 - pert))
    out_ref[...] = jnp.broadcast_to(-total / (2.0 * N), (1, 1))


def _readout(zr, zp, wd):
    return pl.pallas_call(
        _readout_body,
        out_shape=jax.ShapeDtypeStruct((1, 1), jnp.float32),
    )(zr, zp, wd)


def kernel(x, ei, perm, W1a, b1a, g1a, be1a, W2a, b2a, W1b, b1b, g1b, be1b, W2b, b2b, Wd):
    src = ei[0]
    dst = ei[1]
    perm = perm.astype(jnp.int32)

    pad = SLOTS - EPS
    src_r = src.reshape(NSUB, EPS)
    dst_r = dst.reshape(NSUB, EPS)
    zero_pad = jnp.zeros((NSUB, pad), jnp.int32)
    src_pad = jnp.concatenate([src_r, zero_pad], axis=1)
    dummy = N + (jnp.arange(pad, dtype=jnp.int32) % (NPAD - N))
    dst_pad = jnp.concatenate([dst_r, jnp.broadcast_to(dummy, (NSUB, pad))], axis=1)
    dstp = dst_pad.reshape(NSUB, IB, IBLK, CHUNK)
    srcp1 = jnp.stack([src_pad, src_pad + NF]).reshape(2, NSUB, IB, IBLK, CHUNK)
    srcp2 = jnp.stack([src_pad, src_pad + N]).reshape(2, NSUB, IB, IBLK, CHUNK)
    permp = jnp.concatenate([perm, jnp.zeros((NF - N,), jnp.int32)]).reshape(
        NSUB, PCC, PCH)
    zrows = jnp.zeros((RPS, D), jnp.float32)

    zstack = _perm_stack(x, permp)
    z1 = zstack.reshape(2, NF, D)

    a1 = _aggr_pair(zstack, srcp1, dstp, zrows)
    h1 = _mlp_pair(z1, a1, W1a, b1a, g1a, be1a, W2a, b2a)
    a2 = _aggr_pair(h1.reshape(2 * N, D), srcp2, dstp, zrows)
    h2 = _mlp_pair(h1, a2, W1b, b1b, g1b, be1b, W2b, b2b)
    loss = _readout(h2[0], h2[1], Wd)
    return loss.reshape(())


# speedup vs baseline: 3.7584x; 1.1877x over previous
import functools

import jax
import jax.numpy as jnp
from jax import lax
from jax.experimental import pallas as pl
from jax.experimental.pallas import tpu as pltpu
from jax.experimental.pallas import tpu_sc as plsc

N = 10000
D = 128
E = 320000

NSUB = 16
CHUNK = 128
EPS = E // NSUB
IBLK = 16
IB = 10
CH = IB * IBLK
SLOTS = CH * CHUNK
NF = 10240
NPAD = 10240
RPS = NPAD // NSUB
PRW = NF // NSUB
PCH = 64
PCC = PRW // PCH
CPW = 624


def _perm_body(x_hbm, permp, out, perm_v, rows_v, sem):
    c = lax.axis_index("c")
    s = lax.axis_index("s")

    @pl.when(c == 0)
    def _():
        pltpu.sync_copy(x_hbm.at[pl.ds(s * CPW, CPW)], out.at[pl.ds(s * CPW, CPW)])

        @pl.when(s == 0)
        def _():
            pltpu.sync_copy(x_hbm.at[pl.ds(NSUB * CPW, N - NSUB * CPW)],
                            out.at[pl.ds(NSUB * CPW, N - NSUB * CPW)])

    @pl.when(c == 1)
    def _():
        pltpu.sync_copy(permp.at[s], perm_v)
        base = NF + s * PRW
        for cc in range(PCC):
            pltpu.async_copy(x_hbm.at[perm_v.at[cc]], rows_v, sem).wait()
            pltpu.sync_copy(rows_v, out.at[pl.ds(base + cc * PCH, PCH)])


def _perm_stack(x, permp):
    mesh = plsc.VectorSubcoreMesh(core_axis_name="c", subcore_axis_name="s")
    f = functools.partial(
        pl.kernel,
        out_type=jax.ShapeDtypeStruct((2 * NF, D), jnp.float32),
        mesh=mesh,
        scratch_types=[
            pltpu.VMEM((PCC, PCH), jnp.int32),
            pltpu.VMEM((PCH, D), jnp.float32),
            pltpu.SemaphoreType.DMA,
        ],
    )(_perm_body)
    return f(x, permp)


def _aggr_body(zflat, srcp, dstp, zrows, out,
               src_v, dst_v, rows_a, rows_b, acc, sem_a, sem_b):
    c = lax.axis_index("c")
    s = lax.axis_index("s")
    pltpu.sync_copy(zrows, acc.at[pl.ds(s * RPS, RPS)])
    plsc.subcore_barrier()

    def outer(ib, carry):
        pltpu.sync_copy(srcp.at[c, s, ib], src_v)
        pltpu.sync_copy(dstp.at[s, ib], dst_v)
        pltpu.async_copy(zflat.at[src_v.at[0]], rows_a, sem_a)

        def body(p, carry2):
            j = 2 * p
            pltpu.async_copy(zflat.at[src_v.at[j + 1]], rows_b, sem_b)
            pltpu.make_async_copy(zflat.at[src_v.at[j]], rows_a, sem_a).wait()
            pltpu.sync_copy(rows_a, acc.at[dst_v.at[j]], add=True)

            @pl.when(j + 2 < IBLK)
            def _():
                pltpu.async_copy(zflat.at[src_v.at[j + 2]], rows_a, sem_a)

            pltpu.make_async_copy(zflat.at[src_v.at[j + 1]], rows_b, sem_b).wait()
            pltpu.sync_copy(rows_b, acc.at[dst_v.at[j + 1]], add=True)
            return carry2

        return lax.fori_loop(0, IBLK // 2, body, carry)

    lax.fori_loop(0, IB, outer, jnp.int32(0))
    plsc.subcore_barrier()
    pltpu.sync_copy(acc.at[pl.ds(s * RPS, RPS)], out.at[c, pl.ds(s * RPS, RPS)])


def _aggr_pair(zflat, srcp, dstp, zrows):
    mesh = plsc.VectorSubcoreMesh(core_axis_name="c", subcore_axis_name="s")
    f = functools.partial(
        pl.kernel,
        out_type=jax.ShapeDtypeStruct((2, NPAD, D), jnp.float32),
        mesh=mesh,
        scratch_types=[
            pltpu.VMEM((IBLK, CHUNK), jnp.int32),
            pltpu.VMEM((IBLK, CHUNK), jnp.int32),
            pltpu.VMEM((CHUNK, D), jnp.float32),
            pltpu.VMEM((CHUNK, D), jnp.float32),
            pltpu.VMEM_SHARED((NPAD, D), jnp.float32),
            pltpu.SemaphoreType.DMA,
            pltpu.SemaphoreType.DMA,
        ],
    )(_aggr_body)
    return f(zflat, srcp, dstp, zrows)


def _mlp_body(z_ref, a_ref, w1_ref, b1_ref, g_ref, be_ref, w2_ref, b2_ref, out_ref):
    h = z_ref[0] + a_ref[0]
    h = jnp.dot(h, w1_ref[...].T, preferred_element_type=jnp.float32) + b1_ref[...]
    mu = jnp.mean(h, axis=0, keepdims=True)
    var = jnp.mean((h - mu) ** 2, axis=0, keepdims=True)
    h = g_ref[...] * (h - mu) * lax.rsqrt(var + 1e-5) + be_ref[...]
    h = jnp.maximum(h, 0.0)
    h = jnp.dot(h, w2_ref[...].T, preferred_element_type=jnp.float32) + b2_ref[...]
    out_ref[0] = jnp.maximum(h, 0.0)


def _mlp_pair(z2, a2, w1, b1, g, be, w2, b2):
    nz = z2.shape[1]
    zblk = pl.BlockSpec((1, N, D), lambda i: (i, 0, 0))
    wspec = pl.BlockSpec((D, D), lambda i: (0, 0))
    vspec = pl.BlockSpec((D,), lambda i: (0,))
    return pl.pallas_call(
        _mlp_body,
        grid=(2,),
        in_specs=[zblk, zblk, wspec, vspec, vspec, vspec, wspec, vspec],
        out_specs=pl.BlockSpec((1, N, D), lambda i: (i, 0, 0)),
        out_shape=jax.ShapeDtypeStruct((2, N, D), jnp.float32),
    )(z2, a2, w1, b1, g, be, w2, b2)


def _readout_body(zr_ref, zp_ref, wd_ref, out_ref):
    zr = zr_ref[...]
    zp = zp_ref[...]
    s = jax.nn.sigmoid(jnp.mean(zr, axis=0, keepdims=True))
    w = jnp.dot(wd_ref[...].T, s.T, preferred_element_type=jnp.float32)
    real = jax.nn.sigmoid(jnp.dot(zr, w, preferred_element_type=jnp.float32))
    pert = jax.nn.sigmoid(jnp.dot(zp, w, preferred_element_type=jnp.float32))
    eps = 1e-12
    real = jnp.clip(real, eps, 1.0 - eps)
    pert = jnp.clip(pert, eps, 1.0 - eps)
    total = jnp.sum(jnp.log(real)) + jnp.sum(jnp.log(1.0 - pert))
    out_ref[...] = jnp.broadcast_to(-total / (2.0 * N), (1, 1))


def _readout(zr, zp, wd):
    return pl.pallas_call(
        _readout_body,
        out_shape=jax.ShapeDtypeStruct((1, 1), jnp.float32),
    )(zr, zp, wd)


def kernel(x, ei, perm, W1a, b1a, g1a, be1a, W2a, b2a, W1b, b1b, g1b, be1b, W2b, b2b, Wd):
    src = ei[0]
    dst = ei[1]
    perm = perm.astype(jnp.int32)

    pad = SLOTS - EPS
    src_r = src.reshape(NSUB, EPS)
    dst_r = dst.reshape(NSUB, EPS)
    zero_pad = jnp.zeros((NSUB, pad), jnp.int32)
    src_pad = jnp.concatenate([src_r, zero_pad], axis=1)
    dummy = N + (jnp.arange(pad, dtype=jnp.int32) % (NPAD - N))
    dst_pad = jnp.concatenate([dst_r, jnp.broadcast_to(dummy, (NSUB, pad))], axis=1)
    dstp = dst_pad.reshape(NSUB, IB, IBLK, CHUNK)
    srcp1 = jnp.stack([src_pad, src_pad + NF]).reshape(2, NSUB, IB, IBLK, CHUNK)
    srcp2 = jnp.stack([src_pad, src_pad + N]).reshape(2, NSUB, IB, IBLK, CHUNK)
    permp = jnp.concatenate([perm, jnp.zeros((NF - N,), jnp.int32)]).reshape(
        NSUB, PCC, PCH)
    zrows = jnp.zeros((RPS, D), jnp.float32)

    zstack = _perm_stack(x, permp)
    z1 = zstack.reshape(2, NF, D)

    a1 = _aggr_pair(zstack, srcp1, dstp, zrows)
    h1 = _mlp_pair(z1, a1, W1a, b1a, g1a, be1a, W2a, b2a)
    a2 = _aggr_pair(h1.reshape(2 * N, D), srcp2, dstp, zrows)
    h2 = _mlp_pair(h1, a2, W1b, b1b, g1b, be1b, W2b, b2b)
    loss = _readout(h2[0], h2[1], Wd)
    return loss.reshape(())


# speedup vs baseline: 3.7602x; 1.0005x over previous
import functools

import jax
import jax.numpy as jnp
from jax import lax
from jax.experimental import pallas as pl
from jax.experimental.pallas import tpu as pltpu
from jax.experimental.pallas import tpu_sc as plsc

N = 10000
D = 128
E = 320000

NSUB = 16
CHUNK = 128
EPS = E // NSUB
IBLK = 16
IB = 10
CH = IB * IBLK
SLOTS = CH * CHUNK
NF = 10240
NPAD = 10240
RPS = NPAD // NSUB
PRW = NF // NSUB
PCH = 128
PCC = PRW // PCH
CPW = 624


def _perm_body(x_hbm, permp, out, perm_v, rows_a, rows_b, sem_a, sem_b):
    c = lax.axis_index("c")
    s = lax.axis_index("s")

    @pl.when(c == 0)
    def _():
        pltpu.sync_copy(x_hbm.at[pl.ds(s * CPW, CPW)], out.at[pl.ds(s * CPW, CPW)])

        @pl.when(s == 0)
        def _():
            pltpu.sync_copy(x_hbm.at[pl.ds(NSUB * CPW, N - NSUB * CPW)],
                            out.at[pl.ds(NSUB * CPW, N - NSUB * CPW)])

    @pl.when(c == 1)
    def _():
        pltpu.sync_copy(permp.at[s], perm_v)
        base = NF + s * PRW
        bufs = (rows_a, rows_b)
        sems = (sem_a, sem_b)
        for cc in range(PCC):
            pltpu.async_copy(x_hbm.at[perm_v.at[cc]], bufs[cc % 2], sems[cc % 2])
            if cc >= 1:
                pltpu.make_async_copy(x_hbm.at[perm_v.at[cc - 1]],
                                      bufs[(cc - 1) % 2], sems[(cc - 1) % 2]).wait()
                pltpu.sync_copy(bufs[(cc - 1) % 2],
                                out.at[pl.ds(base + (cc - 1) * PCH, PCH)])
        pltpu.make_async_copy(x_hbm.at[perm_v.at[PCC - 1]],
                              bufs[(PCC - 1) % 2], sems[(PCC - 1) % 2]).wait()
        pltpu.sync_copy(bufs[(PCC - 1) % 2],
                        out.at[pl.ds(base + (PCC - 1) * PCH, PCH)])


def _perm_stack(x, permp):
    mesh = plsc.VectorSubcoreMesh(core_axis_name="c", subcore_axis_name="s")
    f = functools.partial(
        pl.kernel,
        out_type=jax.ShapeDtypeStruct((2 * NF, D), jnp.float32),
        mesh=mesh,
        scratch_types=[
            pltpu.VMEM((PCC, PCH), jnp.int32),
            pltpu.VMEM((PCH, D), jnp.float32),
            pltpu.VMEM((PCH, D), jnp.float32),
            pltpu.SemaphoreType.DMA,
            pltpu.SemaphoreType.DMA,
        ],
    )(_perm_body)
    return f(x, permp)


def _aggr_body(zflat, srcp, dstp, zrows, out,
               src_v, dst_v, rows_a, rows_b, acc, sem_a, sem_b):
    c = lax.axis_index("c")
    s = lax.axis_index("s")
    pltpu.sync_copy(zrows, acc.at[pl.ds(s * RPS, RPS)])
    plsc.subcore_barrier()

    def outer(ib, carry):
        pltpu.sync_copy(srcp.at[c, s, ib], src_v)
        pltpu.sync_copy(dstp.at[s, ib], dst_v)
        pltpu.async_copy(zflat.at[src_v.at[0]], rows_a, sem_a)

        def body(p, carry2):
            j = 2 * p
            pltpu.async_copy(zflat.at[src_v.at[j + 1]], rows_b, sem_b)
            pltpu.make_async_copy(zflat.at[src_v.at[j]], rows_a, sem_a).wait()
            pltpu.sync_copy(rows_a, acc.at[dst_v.at[j]], add=True)

            @pl.when(j + 2 < IBLK)
            def _():
                pltpu.async_copy(zflat.at[src_v.at[j + 2]], rows_a, sem_a)

            pltpu.make_async_copy(zflat.at[src_v.at[j + 1]], rows_b, sem_b).wait()
            pltpu.sync_copy(rows_b, acc.at[dst_v.at[j + 1]], add=True)
            return carry2

        return lax.fori_loop(0, IBLK // 2, body, carry)

    lax.fori_loop(0, IB, outer, jnp.int32(0))
    plsc.subcore_barrier()
    pltpu.sync_copy(acc.at[pl.ds(s * RPS, RPS)], out.at[c, pl.ds(s * RPS, RPS)])


def _aggr_pair(zflat, srcp, dstp, zrows):
    mesh = plsc.VectorSubcoreMesh(core_axis_name="c", subcore_axis_name="s")
    f = functools.partial(
        pl.kernel,
        out_type=jax.ShapeDtypeStruct((2, NPAD, D), jnp.float32),
        mesh=mesh,
        scratch_types=[
            pltpu.VMEM((IBLK, CHUNK), jnp.int32),
            pltpu.VMEM((IBLK, CHUNK), jnp.int32),
            pltpu.VMEM((CHUNK, D), jnp.float32),
            pltpu.VMEM((CHUNK, D), jnp.float32),
            pltpu.VMEM_SHARED((NPAD, D), jnp.float32),
            pltpu.SemaphoreType.DMA,
            pltpu.SemaphoreType.DMA,
        ],
    )(_aggr_body)
    return f(zflat, srcp, dstp, zrows)


def _mlp_body(z_ref, a_ref, w1_ref, b1_ref, g_ref, be_ref, w2_ref, b2_ref, out_ref):
    h = z_ref[0] + a_ref[0]
    h = jnp.dot(h, w1_ref[...].T, preferred_element_type=jnp.float32) + b1_ref[...]
    mu = jnp.mean(h, axis=0, keepdims=True)
    var = jnp.mean((h - mu) ** 2, axis=0, keepdims=True)
    h = g_ref[...] * (h - mu) * lax.rsqrt(var + 1e-5) + be_ref[...]
    h = jnp.maximum(h, 0.0)
    h = jnp.dot(h, w2_ref[...].T, preferred_element_type=jnp.float32) + b2_ref[...]
    out_ref[0] = jnp.maximum(h, 0.0)


def _mlp_pair(z2, a2, w1, b1, g, be, w2, b2):
    nz = z2.shape[1]
    zblk = pl.BlockSpec((1, N, D), lambda i: (i, 0, 0))
    wspec = pl.BlockSpec((D, D), lambda i: (0, 0))
    vspec = pl.BlockSpec((D,), lambda i: (0,))
    return pl.pallas_call(
        _mlp_body,
        grid=(2,),
        in_specs=[zblk, zblk, wspec, vspec, vspec, vspec, wspec, vspec],
        out_specs=pl.BlockSpec((1, N, D), lambda i: (i, 0, 0)),
        out_shape=jax.ShapeDtypeStruct((2, N, D), jnp.float32),
    )(z2, a2, w1, b1, g, be, w2, b2)


def _readout_body(zr_ref, zp_ref, wd_ref, out_ref):
    zr = zr_ref[...]
    zp = zp_ref[...]
    s = jax.nn.sigmoid(jnp.mean(zr, axis=0, keepdims=True))
    w = jnp.dot(wd_ref[...].T, s.T, preferred_element_type=jnp.float32)
    real = jax.nn.sigmoid(jnp.dot(zr, w, preferred_element_type=jnp.float32))
    pert = jax.nn.sigmoid(jnp.dot(zp, w, preferred_element_type=jnp.float32))
    eps = 1e-12
    real = jnp.clip(real, eps, 1.0 - eps)
    pert = jnp.clip(pert, eps, 1.0 - eps)
    total = jnp.sum(jnp.log(real)) + jnp.sum(jnp.log(1.0 - pert))
    out_ref[...] = jnp.broadcast_to(-total / (2.0 * N), (1, 1))


def _readout(zr, zp, wd):
    return pl.pallas_call(
        _readout_body,
        out_shape=jax.ShapeDtypeStruct((1, 1), jnp.float32),
    )(zr, zp, wd)


def kernel(x, ei, perm, W1a, b1a, g1a, be1a, W2a, b2a, W1b, b1b, g1b, be1b, W2b, b2b, Wd):
    src = ei[0]
    dst = ei[1]
    perm = perm.astype(jnp.int32)

    pad = SLOTS - EPS
    src_r = src.reshape(NSUB, EPS)
    dst_r = dst.reshape(NSUB, EPS)
    zero_pad = jnp.zeros((NSUB, pad), jnp.int32)
    src_pad = jnp.concatenate([src_r, zero_pad], axis=1)
    dummy = N + (jnp.arange(pad, dtype=jnp.int32) % (NPAD - N))
    dst_pad = jnp.concatenate([dst_r, jnp.broadcast_to(dummy, (NSUB, pad))], axis=1)
    dstp = dst_pad.reshape(NSUB, IB, IBLK, CHUNK)
    srcp1 = jnp.stack([src_pad, src_pad + NF]).reshape(2, NSUB, IB, IBLK, CHUNK)
    srcp2 = jnp.stack([src_pad, src_pad + N]).reshape(2, NSUB, IB, IBLK, CHUNK)
    permp = jnp.concatenate([perm, jnp.zeros((NF - N,), jnp.int32)]).reshape(
        NSUB, PCC, PCH)
    zrows = jnp.zeros((RPS, D), jnp.float32)

    zstack = _perm_stack(x, permp)
    z1 = zstack.reshape(2, NF, D)

    a1 = _aggr_pair(zstack, srcp1, dstp, zrows)
    h1 = _mlp_pair(z1, a1, W1a, b1a, g1a, be1a, W2a, b2a)
    a2 = _aggr_pair(h1.reshape(2 * N, D), srcp2, dstp, zrows)
    h2 = _mlp_pair(h1, a2, W1b, b1b, g1b, be1b, W2b, b2b)
    loss = _readout(h2[0], h2[1], Wd)
    return loss.reshape(())


# speedup vs baseline: 3.7673x; 1.0019x over previous
import functools

import jax
import jax.numpy as jnp
from jax import lax
from jax.experimental import pallas as pl
from jax.experimental.pallas import tpu as pltpu
from jax.experimental.pallas import tpu_sc as plsc

N = 10000
D = 128
E = 320000

NSUB = 16
CHUNK = 128
EPS = E // NSUB
IBLK = 16
IB = 10
CH = IB * IBLK
SLOTS = CH * CHUNK
NF = 10240
NPAD = 10240
RPS = NPAD // NSUB
PRW = NF // NSUB
PCH = 128
PCC = PRW // PCH
CPW = 624


def _perm_body(x_hbm, permp, out, perm_v, rows_v, sem_a):
    c = lax.axis_index("c")
    s = lax.axis_index("s")

    @pl.when(c == 0)
    def _():
        pltpu.sync_copy(x_hbm.at[pl.ds(s * CPW, CPW)], out.at[pl.ds(s * CPW, CPW)])

        @pl.when(s == 0)
        def _():
            pltpu.sync_copy(x_hbm.at[pl.ds(NSUB * CPW, N - NSUB * CPW)],
                            out.at[pl.ds(NSUB * CPW, N - NSUB * CPW)])

    @pl.when(c == 1)
    def _():
        pltpu.sync_copy(permp.at[s], perm_v)
        base = NF + s * PRW
        for cc in range(PCC):
            pltpu.async_copy(x_hbm.at[perm_v.at[cc]],
                             rows_v.at[pl.ds(cc * PCH, PCH)], sem_a)
        for cc in range(PCC):
            pltpu.make_async_copy(x_hbm.at[perm_v.at[cc]],
                                  rows_v.at[pl.ds(cc * PCH, PCH)], sem_a).wait()
        pltpu.sync_copy(rows_v, out.at[pl.ds(base, PRW)])


def _perm_stack(x, permp):
    mesh = plsc.VectorSubcoreMesh(core_axis_name="c", subcore_axis_name="s")
    f = functools.partial(
        pl.kernel,
        out_type=jax.ShapeDtypeStruct((2 * NF, D), jnp.float32),
        mesh=mesh,
        scratch_types=[
            pltpu.VMEM((PCC, PCH), jnp.int32),
            pltpu.VMEM((PRW, D), jnp.float32),
            pltpu.SemaphoreType.DMA,
        ],
    )(_perm_body)
    return f(x, permp)


def _aggr_body(zflat, srcp, dstp, zrows, out,
               src_v, dst_v, rows_a, rows_b, acc, sem_a, sem_b):
    c = lax.axis_index("c")
    s = lax.axis_index("s")
    pltpu.sync_copy(zrows, acc.at[pl.ds(s * RPS, RPS)])
    plsc.subcore_barrier()

    def outer(ib, carry):
        pltpu.sync_copy(srcp.at[c, s, ib], src_v)
        pltpu.sync_copy(dstp.at[s, ib], dst_v)
        pltpu.async_copy(zflat.at[src_v.at[0]], rows_a, sem_a)

        def body(p, carry2):
            j = 2 * p
            pltpu.async_copy(zflat.at[src_v.at[j + 1]], rows_b, sem_b)
            pltpu.make_async_copy(zflat.at[src_v.at[j]], rows_a, sem_a).wait()
            pltpu.sync_copy(rows_a, acc.at[dst_v.at[j]], add=True)

            @pl.when(j + 2 < IBLK)
            def _():
                pltpu.async_copy(zflat.at[src_v.at[j + 2]], rows_a, sem_a)

            pltpu.make_async_copy(zflat.at[src_v.at[j + 1]], rows_b, sem_b).wait()
            pltpu.sync_copy(rows_b, acc.at[dst_v.at[j + 1]], add=True)
            return carry2

        return lax.fori_loop(0, IBLK // 2, body, carry)

    lax.fori_loop(0, IB, outer, jnp.int32(0))
    plsc.subcore_barrier()
    pltpu.sync_copy(acc.at[pl.ds(s * RPS, RPS)], out.at[c, pl.ds(s * RPS, RPS)])


def _aggr_pair(zflat, srcp, dstp, zrows):
    mesh = plsc.VectorSubcoreMesh(core_axis_name="c", subcore_axis_name="s")
    f = functools.partial(
        pl.kernel,
        out_type=jax.ShapeDtypeStruct((2, NPAD, D), jnp.float32),
        mesh=mesh,
        scratch_types=[
            pltpu.VMEM((IBLK, CHUNK), jnp.int32),
            pltpu.VMEM((IBLK, CHUNK), jnp.int32),
            pltpu.VMEM((CHUNK, D), jnp.float32),
            pltpu.VMEM((CHUNK, D), jnp.float32),
            pltpu.VMEM_SHARED((NPAD, D), jnp.float32),
            pltpu.SemaphoreType.DMA,
            pltpu.SemaphoreType.DMA,
        ],
    )(_aggr_body)
    return f(zflat, srcp, dstp, zrows)


def _mlp_body(z_ref, a_ref, w1_ref, b1_ref, g_ref, be_ref, w2_ref, b2_ref, out_ref):
    h = z_ref[0] + a_ref[0]
    h = jnp.dot(h, w1_ref[...].T, preferred_element_type=jnp.float32) + b1_ref[...]
    mu = jnp.mean(h, axis=0, keepdims=True)
    var = jnp.mean((h - mu) ** 2, axis=0, keepdims=True)
    h = g_ref[...] * (h - mu) * lax.rsqrt(var + 1e-5) + be_ref[...]
    h = jnp.maximum(h, 0.0)
    h = jnp.dot(h, w2_ref[...].T, preferred_element_type=jnp.float32) + b2_ref[...]
    out_ref[0] = jnp.maximum(h, 0.0)


def _mlp_pair(z2, a2, w1, b1, g, be, w2, b2):
    nz = z2.shape[1]
    zblk = pl.BlockSpec((1, N, D), lambda i: (i, 0, 0))
    wspec = pl.BlockSpec((D, D), lambda i: (0, 0))
    vspec = pl.BlockSpec((D,), lambda i: (0,))
    return pl.pallas_call(
        _mlp_body,
        grid=(2,),
        in_specs=[zblk, zblk, wspec, vspec, vspec, vspec, wspec, vspec],
        out_specs=pl.BlockSpec((1, N, D), lambda i: (i, 0, 0)),
        out_shape=jax.ShapeDtypeStruct((2, N, D), jnp.float32),
    )(z2, a2, w1, b1, g, be, w2, b2)


def _readout_body(zr_ref, zp_ref, wd_ref, out_ref):
    zr = zr_ref[...]
    zp = zp_ref[...]
    s = jax.nn.sigmoid(jnp.mean(zr, axis=0, keepdims=True))
    w = jnp.dot(wd_ref[...].T, s.T, preferred_element_type=jnp.float32)
    real = jax.nn.sigmoid(jnp.dot(zr, w, preferred_element_type=jnp.float32))
    pert = jax.nn.sigmoid(jnp.dot(zp, w, preferred_element_type=jnp.float32))
    eps = 1e-12
    real = jnp.clip(real, eps, 1.0 - eps)
    pert = jnp.clip(pert, eps, 1.0 - eps)
    total = jnp.sum(jnp.log(real)) + jnp.sum(jnp.log(1.0 - pert))
    out_ref[...] = jnp.broadcast_to(-total / (2.0 * N), (1, 1))


def _readout(zr, zp, wd):
    return pl.pallas_call(
        _readout_body,
        out_shape=jax.ShapeDtypeStruct((1, 1), jnp.float32),
    )(zr, zp, wd)


def kernel(x, ei, perm, W1a, b1a, g1a, be1a, W2a, b2a, W1b, b1b, g1b, be1b, W2b, b2b, Wd):
    src = ei[0]
    dst = ei[1]
    perm = perm.astype(jnp.int32)

    pad = SLOTS - EPS
    src_r = src.reshape(NSUB, EPS)
    dst_r = dst.reshape(NSUB, EPS)
    zero_pad = jnp.zeros((NSUB, pad), jnp.int32)
    src_pad = jnp.concatenate([src_r, zero_pad], axis=1)
    dummy = N + (jnp.arange(pad, dtype=jnp.int32) % (NPAD - N))
    dst_pad = jnp.concatenate([dst_r, jnp.broadcast_to(dummy, (NSUB, pad))], axis=1)
    dstp = dst_pad.reshape(NSUB, IB, IBLK, CHUNK)
    srcp1 = jnp.stack([src_pad, src_pad + NF]).reshape(2, NSUB, IB, IBLK, CHUNK)
    srcp2 = jnp.stack([src_pad, src_pad + N]).reshape(2, NSUB, IB, IBLK, CHUNK)
    permp = jnp.concatenate([perm, jnp.zeros((NF - N,), jnp.int32)]).reshape(
        NSUB, PCC, PCH)
    zrows = jnp.zeros((RPS, D), jnp.float32)

    zstack = _perm_stack(x, permp)
    z1 = zstack.reshape(2, NF, D)

    a1 = _aggr_pair(zstack, srcp1, dstp, zrows)
    h1 = _mlp_pair(z1, a1, W1a, b1a, g1a, be1a, W2a, b2a)
    a2 = _aggr_pair(h1.reshape(2 * N, D), srcp2, dstp, zrows)
    h2 = _mlp_pair(h1, a2, W1b, b1b, g1b, be1b, W2b, b2b)
    loss = _readout(h2[0], h2[1], Wd)
    return loss.reshape(())


# speedup vs baseline: 3.7686x; 1.0003x over previous
import functools

import jax
import jax.numpy as jnp
from jax import lax
from jax.experimental import pallas as pl
from jax.experimental.pallas import tpu as pltpu
from jax.experimental.pallas import tpu_sc as plsc

N = 10000
D = 128
E = 320000

NSUB = 16
CHUNK = 128
EPS = E // NSUB
IBLK = 16
IB = 10
CH = IB * IBLK
SLOTS = CH * CHUNK
NF = 10240
NPAD = 10240
RPS = NPAD // NSUB
PRW = NF // NSUB
PCH = 128
PCC = PRW // PCH
CPW = 624


def _perm_body(x_hbm, permp, out, perm_v, rows_v, sem_a):
    c = lax.axis_index("c")
    s = lax.axis_index("s")

    @pl.when(c == 0)
    def _():
        pltpu.sync_copy(x_hbm.at[pl.ds(s * CPW, CPW)], out.at[pl.ds(s * CPW, CPW)])

        @pl.when(s == 0)
        def _():
            pltpu.sync_copy(x_hbm.at[pl.ds(NSUB * CPW, N - NSUB * CPW)],
                            out.at[pl.ds(NSUB * CPW, N - NSUB * CPW)])

    @pl.when(c == 1)
    def _():
        pltpu.sync_copy(permp.at[s], perm_v)
        base = NF + s * PRW
        pltpu.async_copy(x_hbm.at[perm_v], rows_v, sem_a).wait()
        pltpu.sync_copy(rows_v, out.at[pl.ds(base, PRW)])


def _perm_stack(x, permp):
    mesh = plsc.VectorSubcoreMesh(core_axis_name="c", subcore_axis_name="s")
    f = functools.partial(
        pl.kernel,
        out_type=jax.ShapeDtypeStruct((2 * NF, D), jnp.float32),
        mesh=mesh,
        scratch_types=[
            pltpu.VMEM((PRW,), jnp.int32),
            pltpu.VMEM((PRW, D), jnp.float32),
            pltpu.SemaphoreType.DMA,
        ],
    )(_perm_body)
    return f(x, permp)


def _aggr_body(zflat, srcp, dstp, zrows, out,
               src_v, dst_v, rows_a, rows_b, acc, sem_a, sem_b):
    c = lax.axis_index("c")
    s = lax.axis_index("s")
    pltpu.sync_copy(zrows, acc.at[pl.ds(s * RPS, RPS)])
    plsc.subcore_barrier()

    def outer(ib, carry):
        pltpu.sync_copy(srcp.at[c, s, ib], src_v)
        pltpu.sync_copy(dstp.at[s, ib], dst_v)
        pltpu.async_copy(zflat.at[src_v.at[0]], rows_a, sem_a)

        def body(p, carry2):
            j = 2 * p
            pltpu.async_copy(zflat.at[src_v.at[j + 1]], rows_b, sem_b)
            pltpu.make_async_copy(zflat.at[src_v.at[j]], rows_a, sem_a).wait()
            pltpu.sync_copy(rows_a, acc.at[dst_v.at[j]], add=True)

            @pl.when(j + 2 < IBLK)
            def _():
                pltpu.async_copy(zflat.at[src_v.at[j + 2]], rows_a, sem_a)

            pltpu.make_async_copy(zflat.at[src_v.at[j + 1]], rows_b, sem_b).wait()
            pltpu.sync_copy(rows_b, acc.at[dst_v.at[j + 1]], add=True)
            return carry2

        return lax.fori_loop(0, IBLK // 2, body, carry)

    lax.fori_loop(0, IB, outer, jnp.int32(0))
    plsc.subcore_barrier()
    pltpu.sync_copy(acc.at[pl.ds(s * RPS, RPS)], out.at[c, pl.ds(s * RPS, RPS)])


def _aggr_pair(zflat, srcp, dstp, zrows):
    mesh = plsc.VectorSubcoreMesh(core_axis_name="c", subcore_axis_name="s")
    f = functools.partial(
        pl.kernel,
        out_type=jax.ShapeDtypeStruct((2, NPAD, D), jnp.float32),
        mesh=mesh,
        scratch_types=[
            pltpu.VMEM((IBLK, CHUNK), jnp.int32),
            pltpu.VMEM((IBLK, CHUNK), jnp.int32),
            pltpu.VMEM((CHUNK, D), jnp.float32),
            pltpu.VMEM((CHUNK, D), jnp.float32),
            pltpu.VMEM_SHARED((NPAD, D), jnp.float32),
            pltpu.SemaphoreType.DMA,
            pltpu.SemaphoreType.DMA,
        ],
    )(_aggr_body)
    return f(zflat, srcp, dstp, zrows)


def _mlp_body(z_ref, a_ref, w1_ref, b1_ref, g_ref, be_ref, w2_ref, b2_ref, out_ref):
    h = z_ref[0] + a_ref[0]
    h = jnp.dot(h, w1_ref[...].T, preferred_element_type=jnp.float32) + b1_ref[...]
    mu = jnp.mean(h, axis=0, keepdims=True)
    var = jnp.mean((h - mu) ** 2, axis=0, keepdims=True)
    h = g_ref[...] * (h - mu) * lax.rsqrt(var + 1e-5) + be_ref[...]
    h = jnp.maximum(h, 0.0)
    h = jnp.dot(h, w2_ref[...].T, preferred_element_type=jnp.float32) + b2_ref[...]
    out_ref[0] = jnp.maximum(h, 0.0)


def _mlp_pair(z2, a2, w1, b1, g, be, w2, b2):
    nz = z2.shape[1]
    zblk = pl.BlockSpec((1, N, D), lambda i: (i, 0, 0))
    wspec = pl.BlockSpec((D, D), lambda i: (0, 0))
    vspec = pl.BlockSpec((D,), lambda i: (0,))
    return pl.pallas_call(
        _mlp_body,
        grid=(2,),
        in_specs=[zblk, zblk, wspec, vspec, vspec, vspec, wspec, vspec],
        out_specs=pl.BlockSpec((1, N, D), lambda i: (i, 0, 0)),
        out_shape=jax.ShapeDtypeStruct((2, N, D), jnp.float32),
    )(z2, a2, w1, b1, g, be, w2, b2)


def _readout_body(zr_ref, zp_ref, wd_ref, out_ref):
    zr = zr_ref[...]
    zp = zp_ref[...]
    s = jax.nn.sigmoid(jnp.mean(zr, axis=0, keepdims=True))
    w = jnp.dot(wd_ref[...].T, s.T, preferred_element_type=jnp.float32)
    real = jax.nn.sigmoid(jnp.dot(zr, w, preferred_element_type=jnp.float32))
    pert = jax.nn.sigmoid(jnp.dot(zp, w, preferred_element_type=jnp.float32))
    eps = 1e-12
    real = jnp.clip(real, eps, 1.0 - eps)
    pert = jnp.clip(pert, eps, 1.0 - eps)
    total = jnp.sum(jnp.log(real)) + jnp.sum(jnp.log(1.0 - pert))
    out_ref[...] = jnp.broadcast_to(-total / (2.0 * N), (1, 1))


def _readout(zr, zp, wd):
    return pl.pallas_call(
        _readout_body,
        out_shape=jax.ShapeDtypeStruct((1, 1), jnp.float32),
    )(zr, zp, wd)


def kernel(x, ei, perm, W1a, b1a, g1a, be1a, W2a, b2a, W1b, b1b, g1b, be1b, W2b, b2b, Wd):
    src = ei[0]
    dst = ei[1]
    perm = perm.astype(jnp.int32)

    pad = SLOTS - EPS
    src_r = src.reshape(NSUB, EPS)
    dst_r = dst.reshape(NSUB, EPS)
    zero_pad = jnp.zeros((NSUB, pad), jnp.int32)
    src_pad = jnp.concatenate([src_r, zero_pad], axis=1)
    dummy = N + (jnp.arange(pad, dtype=jnp.int32) % (NPAD - N))
    dst_pad = jnp.concatenate([dst_r, jnp.broadcast_to(dummy, (NSUB, pad))], axis=1)
    dstp = dst_pad.reshape(NSUB, IB, IBLK, CHUNK)
    srcp1 = jnp.stack([src_pad, src_pad + NF]).reshape(2, NSUB, IB, IBLK, CHUNK)
    srcp2 = jnp.stack([src_pad, src_pad + N]).reshape(2, NSUB, IB, IBLK, CHUNK)
    permp = jnp.concatenate([perm, jnp.zeros((NF - N,), jnp.int32)]).reshape(
        NSUB, PRW)
    zrows = jnp.zeros((RPS, D), jnp.float32)

    zstack = _perm_stack(x, permp)
    z1 = zstack.reshape(2, NF, D)

    a1 = _aggr_pair(zstack, srcp1, dstp, zrows)
    h1 = _mlp_pair(z1, a1, W1a, b1a, g1a, be1a, W2a, b2a)
    a2 = _aggr_pair(h1.reshape(2 * N, D), srcp2, dstp, zrows)
    h2 = _mlp_pair(h1, a2, W1b, b1b, g1b, be1b, W2b, b2b)
    loss = _readout(h2[0], h2[1], Wd)
    return loss.reshape(())
